# baseline (speedup 1.0000x reference)
import functools

import jax
import jax.numpy as jnp
from jax import lax
from jax.experimental import pallas as pl
from jax.experimental.pallas import tpu as pltpu
from jax.experimental.pallas import tpu_sc as plsc

_NC = 2
_NS = 16
_NW = _NC * _NS
_LANES = 16
_IDX_CHUNK = 128
_PACK = 4
_BLK = 1024


def _make_sc_repack(rows, dim):
    mesh = plsc.VectorSubcoreMesh(core_axis_name="c", subcore_axis_name="s")
    row_w = dim * _PACK
    n_full = rows // _BLK
    rem = rows - n_full * _BLK
    rem_al = (rem // _IDX_CHUNK) * _IDX_CHUNK
    rem_un = rem - rem_al
    blocks_per_w = (n_full + _NW - 1) // _NW
    out_rows_blk = _BLK // _PACK

    @functools.partial(
        pl.kernel,
        mesh=mesh,
        compiler_params=pltpu.CompilerParams(needs_layout_passes=False),
        out_type=jax.ShapeDtypeStruct((rows // _PACK, row_w), jnp.float32),
        scratch_types=[
            pltpu.VMEM((dim, _BLK), jnp.float32),
            pltpu.VMEM((dim, max(rem_un, _PACK)), jnp.float32),
            pltpu.VMEM((out_rows_blk, row_w), jnp.float32),
        ],
    )
    def repack_kernel(tabt_hbm, tail_hbm, out_hbm, in_v, tail_v, out_v):
        wid = lax.axis_index("s") * _NC + lax.axis_index("c")
        lane_iota = lax.iota(jnp.int32, _LANES)

        def transform(src, n_out_rows):
            def trow(r, _):
                for cc in range(0, row_w, _LANES):
                    m, c0 = cc // dim, cc % dim
                    u = plsc.load_gather(
                        src, [c0 + lane_iota,
                              jnp.full((_LANES,), 0, jnp.int32)
                              + (r * _PACK + m)])
                    out_v[r, pl.ds(cc, _LANES)] = u
                return 0
            lax.fori_loop(0, n_out_rows, trow, 0)

        def do_block(bi, _):
            b = wid + _NW * bi

            @pl.when(b < n_full)
            def _():
                pltpu.sync_copy(tabt_hbm.at[:, pl.ds(b * _BLK, _BLK)], in_v)
                transform(in_v, out_rows_blk)
                pltpu.sync_copy(out_v,
                                out_hbm.at[pl.ds(b * out_rows_blk,
                                                 out_rows_blk)])
            return 0

        lax.fori_loop(0, blocks_per_w, do_block, 0)

        if rem_al:
            @pl.when(wid == 0)
            def _():
                pltpu.sync_copy(
                    tabt_hbm.at[:, pl.ds(n_full * _BLK, rem_al)],
                    in_v.at[:, pl.ds(0, rem_al)])
                transform(in_v, rem_al // _PACK)
                pltpu.sync_copy(
                    out_v.at[pl.ds(0, rem_al // _PACK)],
                    out_hbm.at[pl.ds(n_full * out_rows_blk,
                                     rem_al // _PACK)])

        if rem_un:
            @pl.when(wid == 1)
            def _():
                pltpu.sync_copy(tail_hbm, tail_v)
                transform(tail_v, rem_un // _PACK)
                pltpu.sync_copy(
                    out_v.at[pl.ds(0, rem_un // _PACK)],
                    out_hbm.at[pl.ds((rows - rem_un) // _PACK,
                                     rem_un // _PACK)])

    return repack_kernel


def _make_sc_gather(batch, dim):
    assert batch % (8 * _NW) == 0
    assert dim == 2 * _LANES
    b_per_w = batch // _NW
    half = b_per_w // 2
    mesh = plsc.VectorSubcoreMesh(core_axis_name="c", subcore_axis_name="s")
    row_w = dim * _PACK

    @functools.partial(
        pl.kernel,
        mesh=mesh,
        compiler_params=pltpu.CompilerParams(needs_layout_passes=False),
        out_type=jax.ShapeDtypeStruct((batch,), jnp.float32),
        scratch_types=[
            pltpu.VMEM((b_per_w,), jnp.int32),
            pltpu.VMEM((b_per_w,), jnp.int32),
            pltpu.VMEM((b_per_w,), jnp.int32),
            pltpu.VMEM((b_per_w,), jnp.int32),
            pltpu.VMEM((half, row_w), jnp.float32),
            pltpu.VMEM((half, row_w), jnp.float32),
            pltpu.VMEM((b_per_w,), jnp.float32),
            pltpu.SemaphoreType.DMA,
            pltpu.SemaphoreType.DMA,
        ],
    )
    def sc_kernel(uid_hbm, iid_hbm, utab_hbm, itab_hbm, out_hbm,
                  uidx_v, iidx_v, uq_v, iq_v, urows_v, irows_v, out_v,
                  usem, isem):
        wid = lax.axis_index("s") * _NC + lax.axis_index("c")
        base = wid * b_per_w

        pltpu.sync_copy(uid_hbm.at[pl.ds(base, b_per_w)], uidx_v)
        pltpu.sync_copy(iid_hbm.at[pl.ds(base, b_per_w)], iidx_v)
        for t in range(b_per_w // _LANES):
            sl = pl.ds(t * _LANES, _LANES)
            uq_v[sl] = jax.lax.shift_right_logical(uidx_v[sl], 2)
            iq_v[sl] = jax.lax.shift_right_logical(iidx_v[sl], 2)

        lane_iota = lax.iota(jnp.int32, _LANES)

        for h in range(2):
            for j in range(half // _IDX_CHUNK):
                isl = pl.ds(h * half + j * _IDX_CHUNK, _IDX_CHUNK)
                dsl = pl.ds(j * _IDX_CHUNK, _IDX_CHUNK)
                pltpu.async_copy(utab_hbm.at[uq_v.at[isl]],
                                 urows_v.at[dsl], usem)
                pltpu.async_copy(itab_hbm.at[iq_v.at[isl]],
                                 irows_v.at[dsl], isem)
            pltpu.make_async_copy(utab_hbm.at[pl.ds(0, half)], urows_v,
                                  usem).wait()
            pltpu.make_async_copy(itab_hbm.at[pl.ds(0, half)], irows_v,
                                  isem).wait()

            def body(g, _):
                off = g * _LANES
                rvec = off + lane_iota
                ucol = (uidx_v[pl.ds(h * half + off, _LANES)] & 3) * dim
                icol = (iidx_v[pl.ds(h * half + off, _LANES)] & 3) * dim
                acc = jnp.zeros((_LANES,), jnp.float32)
                for k in range(dim):
                    u = plsc.load_gather(urows_v, [rvec, ucol + k])
                    i = plsc.load_gather(irows_v, [rvec, icol + k])
                    acc = acc + u * i
                out_v[pl.ds(h * half + off, _LANES)] = acc
                return 0

            lax.fori_loop(0, half // _LANES, body, 0)

        pltpu.sync_copy(out_v, out_hbm.at[pl.ds(base, b_per_w)])

    return sc_kernel


@jax.jit
def kernel(user_id, item_id, user_table, item_table):
    batch = user_id.shape[0]
    rows, dim = user_table.shape
    repack = _make_sc_repack(rows, dim)
    gather = _make_sc_gather(batch, dim)
    tail = rows - (rows // _IDX_CHUNK) * _IDX_CHUNK
    ulin = repack(user_table.T, user_table.T[:, rows - tail:])
    ilin = repack(item_table.T, item_table.T[:, rows - tail:])
    return gather(user_id, item_id, ulin, ilin)


# speedup vs baseline: 2.0193x; 2.0193x over previous
import functools

import jax
import jax.numpy as jnp
from jax import lax
from jax.experimental import pallas as pl
from jax.experimental.pallas import tpu as pltpu
from jax.experimental.pallas import tpu_sc as plsc

_NC = 2
_NS = 16
_NW = _NC * _NS
_LANES = 16
_IDX_CHUNK = 128


def _make_sc_kernel(batch, dim):
    assert batch % (8 * _NW) == 0
    assert dim == 2 * _LANES
    b_per_w = batch // _NW
    n_chunks = b_per_w // _IDX_CHUNK
    mesh = plsc.VectorSubcoreMesh(core_axis_name="c", subcore_axis_name="s")

    @functools.partial(
        pl.kernel,
        mesh=mesh,
        compiler_params=pltpu.CompilerParams(use_tc_tiling_on_sc=False,
                                             needs_layout_passes=False),
        out_type=jax.ShapeDtypeStruct((batch,), jnp.float32),
        scratch_types=[
            pltpu.VMEM((b_per_w,), jnp.int32),
            pltpu.VMEM((b_per_w,), jnp.int32),
            pltpu.VMEM((b_per_w, dim), jnp.float32),
            pltpu.VMEM((b_per_w, dim), jnp.float32),
            pltpu.VMEM((b_per_w,), jnp.float32),
            pltpu.VMEM((_LANES * _LANES,), jnp.float32),
            pltpu.SemaphoreType.DMA,
            pltpu.SemaphoreType.DMA,
        ],
    )
    def sc_kernel(uid_hbm, iid_hbm, utab_hbm, itab_hbm, out_hbm,
                  uidx_v, iidx_v, urows_v, irows_v, out_v, stage_v,
                  usem, isem):
        wid = lax.axis_index("s") * _NC + lax.axis_index("c")
        base = wid * b_per_w

        pltpu.sync_copy(uid_hbm.at[pl.ds(base, b_per_w)], uidx_v)
        pltpu.sync_copy(iid_hbm.at[pl.ds(base, b_per_w)], iidx_v)

        for j in range(n_chunks):
            sl = pl.ds(j * _IDX_CHUNK, _IDX_CHUNK)
            pltpu.async_copy(utab_hbm.at[uidx_v.at[sl]], urows_v.at[sl], usem)
            pltpu.async_copy(itab_hbm.at[iidx_v.at[sl]], irows_v.at[sl], isem)
        pltpu.make_async_copy(utab_hbm.at[pl.ds(0, b_per_w)], urows_v,
                              usem).wait()
        pltpu.make_async_copy(itab_hbm.at[pl.ds(0, b_per_w)], irows_v,
                              isem).wait()

        lane_iota = lax.iota(jnp.int32, _LANES)
        col_base = lane_iota * _LANES

        def body(g, _):
            row0 = g * _LANES
            for rl in range(_LANES):
                u0 = urows_v[row0 + rl, pl.ds(0, _LANES)]
                i0 = irows_v[row0 + rl, pl.ds(0, _LANES)]
                u1 = urows_v[row0 + rl, pl.ds(_LANES, _LANES)]
                i1 = irows_v[row0 + rl, pl.ds(_LANES, _LANES)]
                stage_v[pl.ds(rl * _LANES, _LANES)] = u0 * i0 + u1 * i1
            acc = plsc.load_gather(stage_v, [col_base])
            for c in range(1, _LANES):
                acc = acc + plsc.load_gather(stage_v, [col_base + c])
            out_v[pl.ds(row0, _LANES)] = acc
            return 0

        lax.fori_loop(0, b_per_w // _LANES, body, 0)

        pltpu.sync_copy(out_v, out_hbm.at[pl.ds(base, b_per_w)])

    return sc_kernel


@jax.jit
def kernel(user_id, item_id, user_table, item_table):
    batch = user_id.shape[0]
    dim = user_table.shape[1]
    fn = _make_sc_kernel(batch, dim)
    return fn(user_id, item_id, user_table, item_table)
